# baseline (speedup 1.0000x reference)
import functools

import jax
import jax.numpy as jnp
from jax import lax
from jax.experimental import pallas as pl
from jax.experimental.pallas import tpu as pltpu
from jax.experimental.pallas import tpu_sc as plsc


_BN = 1000


def _dense(x, w, b, relu):
    n, k = x.shape
    m = w.shape[1]

    def mm(x_ref, w_ref, b_ref, o_ref):
        acc = jnp.dot(x_ref[...], w_ref[...], preferred_element_type=jnp.float32)
        acc = acc + b_ref[...]
        if relu:
            acc = jnp.maximum(acc, 0.0)
        o_ref[...] = acc

    return pl.pallas_call(
        mm,
        grid=(n // _BN,),
        in_specs=[
            pl.BlockSpec((_BN, k), lambda i: (i, 0)),
            pl.BlockSpec((k, m), lambda i: (0, 0)),
            pl.BlockSpec((1, m), lambda i: (0, 0)),
        ],
        out_specs=pl.BlockSpec((_BN, m), lambda i: (i, 0)),
        out_shape=jax.ShapeDtypeStruct((n, m), jnp.float32),
    )(x, w, b.reshape(1, m))


def _combine(x, wx, b, hn, wn, relu):
    n, k = x.shape
    m = wx.shape[1]
    kn = hn.shape[1]

    def mm(x_ref, wx_ref, b_ref, hn_ref, wn_ref, o_ref):
        acc = jnp.dot(x_ref[...], wx_ref[...], preferred_element_type=jnp.float32)
        acc = acc + jnp.dot(hn_ref[...], wn_ref[...],
                            preferred_element_type=jnp.float32)
        acc = acc + b_ref[...]
        if relu:
            acc = jnp.maximum(acc, 0.0)
        o_ref[...] = acc

    return pl.pallas_call(
        mm,
        grid=(n // _BN,),
        in_specs=[
            pl.BlockSpec((_BN, k), lambda i: (i, 0)),
            pl.BlockSpec((k, m), lambda i: (0, 0)),
            pl.BlockSpec((1, m), lambda i: (0, 0)),
            pl.BlockSpec((_BN, kn), lambda i: (i, 0)),
            pl.BlockSpec((kn, m), lambda i: (0, 0)),
        ],
        out_specs=pl.BlockSpec((_BN, m), lambda i: (i, 0)),
        out_shape=jax.ShapeDtypeStruct((n, m), jnp.float32),
    )(x, wx, b.reshape(1, m), hn, wn)


_NC, _NS = 2, 16
_NW = _NC * _NS
_NPT = 320
_BMUL, _BSH = 6554, 21
_CE = 10000
_BLK = 512
_G = 128


def _mesh():
    return plsc.VectorSubcoreMesh(core_axis_name="c", subcore_axis_name="s")


def _take16(v, idx):
    return lax.gather(
        v, idx[:, None],
        lax.GatherDimensionNumbers(offset_dims=(), collapsed_slice_dims=(0,),
                                   start_index_map=(0,)),
        slice_sizes=(1,),
        mode=lax.GatherScatterMode.PROMISE_IN_BOUNDS)


def _scan_edges(src, dst):
    e = src.shape[0]
    ept = e // _NW
    assert ept % _CE == 0 and _CE % 16 == 0
    lcap = ((ept + _BLK - 1) // _BLK) * _BLK
    ltot = _NW * _NW * lcap

    @functools.partial(
        pl.kernel,
        out_type=(jax.ShapeDtypeStruct((ltot,), jnp.int32),
                  jax.ShapeDtypeStruct((ltot,), jnp.int32),
                  jax.ShapeDtypeStruct((_NW * _NW,), jnp.int32)),
        mesh=_mesh(),
        scratch_types=[
            pltpu.VMEM((_NW, _BLK + 16), jnp.int32),
            pltpu.VMEM((_NW, _BLK + 16), jnp.int32),
            pltpu.VMEM((_CE,), jnp.int32),
            pltpu.VMEM((_CE,), jnp.int32),
            pltpu.VMEM((_NW + 16,), jnp.int32),
            pltpu.SMEM((_NW,), jnp.int32),
            pltpu.SMEM((_NW,), jnp.int32),
            pltpu.SemaphoreType.DMA,
        ],
    )
    def scank(src_hbm, dst_hbm, ls_hbm, ld_hbm, cnt_hbm, bs, bd, srcb, dstb,
              cntv, pos, nblk, fsem):
        t = lax.axis_index("s") * _NC + lax.axis_index("c")
        ebase = t * ept
        for b in range(_NW):
            pos[b] = jnp.int32(0)
            nblk[b] = jnp.int32(0)

        def chunk(c, _):
            pltpu.sync_copy(src_hbm.at[pl.ds(ebase + c * _CE, _CE)], srcb)
            pltpu.sync_copy(dst_hbm.at[pl.ds(ebase + c * _CE, _CE)], dstb)

            def grp(i, _):
                s16 = srcb[pl.ds(i * 16, 16)]
                d16 = dstb[pl.ds(i * 16, 16)]
                b16 = (d16 * _BMUL) >> _BSH
                iota16g = lax.iota(jnp.int32, 16)
                for l in range(16):
                    bb = b16[l]
                    c0 = pos[bb]
                    sl = pl.ds((c0 >> 4) * 16, 16)
                    oh = iota16g == (c0 & 15)
                    bs[bb, sl] = jnp.where(oh, jnp.full((16,), s16[l],
                                                        jnp.int32), bs[bb, sl])
                    bd[bb, sl] = jnp.where(oh, jnp.full((16,), d16[l],
                                                        jnp.int32), bd[bb, sl])
                    pos[bb] = c0 + 1

                    def flush(x):
                        base = (t * _NW + bb) * lcap + nblk[bb] * _BLK
                        pltpu.sync_copy(bs.at[bb, pl.ds(0, _BLK)],
                                        ls_hbm.at[pl.ds(base, _BLK)])
                        pltpu.sync_copy(bd.at[bb, pl.ds(0, _BLK)],
                                        ld_hbm.at[pl.ds(base, _BLK)])
                        nblk[bb] = nblk[bb] + 1
                        pos[bb] = jnp.int32(0)
                        return x

                    lax.cond(c0 + 1 >= _BLK, flush, lambda x: x, jnp.int32(0))
                return 0

            lax.fori_loop(0, _CE // 16, grp, 0)
            return 0

        lax.fori_loop(0, ept // _CE, chunk, 0)

        zero16 = jnp.zeros((16,), jnp.int32)

        iota16 = lax.iota(jnp.int32, 16)

        def final_flush(b, _):
            c0 = pos[b]
            dummy = jnp.full((16,), (b + 1) * _NPT, jnp.int32)
            for j in range(_BLK // 16):
                m = (iota16 + j * 16) >= c0
                sl = pl.ds(j * 16, 16)
                bs[b, sl] = jnp.where(m, zero16, bs[b, sl])
                bd[b, sl] = jnp.where(m, dummy, bd[b, sl])
            base = (t * _NW + b) * lcap + nblk[b] * _BLK
            pltpu.sync_copy(bs.at[b, pl.ds(0, _BLK)],
                            ls_hbm.at[pl.ds(base, _BLK)])
            pltpu.sync_copy(bd.at[b, pl.ds(0, _BLK)],
                            ld_hbm.at[pl.ds(base, _BLK)])
            return 0

        lax.fori_loop(0, _NW, final_flush, 0)

        for b in range(_NW - 1, -1, -1):
            cntv[pl.ds(b, 16)] = jnp.full((16,), nblk[b] * _BLK + pos[b],
                                          jnp.int32)
        pltpu.sync_copy(cntv.at[pl.ds(0, _NW)],
                        cnt_hbm.at[pl.ds(t * _NW, _NW)])

    return scank(src, dst), lcap


def _merge(x, ls, ld, counts, lcap):
    n, d = x.shape
    npad = _NW * _NPT
    g = 128
    assert d == 128
    kmax = (_NW * lcap) // g

    @functools.partial(
        pl.kernel,
        out_type=jax.ShapeDtypeStruct((npad, d), jnp.float32),
        mesh=_mesh(),
        scratch_types=[
            pltpu.VMEM((_NPT + 1, d), jnp.float32),
            pltpu.VMEM((g, d), jnp.float32),
            pltpu.VMEM((g, d), jnp.float32),
            pltpu.VMEM((g,), jnp.int32),
            pltpu.VMEM((g,), jnp.int32),
            pltpu.VMEM((g,), jnp.int32),
            pltpu.VMEM((g,), jnp.int32),
            pltpu.VMEM((g,), jnp.int32),
            pltpu.VMEM((g,), jnp.int32),
            pltpu.VMEM((g,), jnp.int32),
            pltpu.VMEM((g,), jnp.int32),
            pltpu.VMEM((kmax,), jnp.int32),
            pltpu.VMEM((_NW * _NW,), jnp.int32),
            pltpu.SemaphoreType.DMA,
            pltpu.SemaphoreType.DMA,
            pltpu.SemaphoreType.DMA,
            pltpu.SemaphoreType.DMA,
            pltpu.SemaphoreType.DMA,
            pltpu.SemaphoreType.DMA,
        ],
    )
    def mergek(x_hbm, ls_hbm, ld_hbm, cnt_hbm, out_hbm, acc, rb0, rb1,
               lv0, lv1, lv2, lv3, dv0, dv1, dv2, dv3, table, cv,
               gs0, gs1, ls0, ls1, ls2, ls3):
        rbs = (rb0, rb1)
        lvs = (lv0, lv1, lv2, lv3)
        dvs = (dv0, dv1, dv2, dv3)
        gsems = (gs0, gs1)
        lsems = (ls0, ls1, ls2, ls3)
        b = lax.axis_index("s") * _NC + lax.axis_index("c")
        lo = b * _NPT
        zero16 = jnp.zeros((16,), jnp.float32)
        iota16 = lax.iota(jnp.int32, 16)

        def zero_row(r, _):
            for j in range(d // 16):
                acc[r, pl.ds(j * 16, 16)] = zero16
            return 0

        lax.fori_loop(0, _NPT + 1, zero_row, 0)
        pltpu.sync_copy(cnt_hbm, cv)
        rotb = (iota16 + (b & 15)) & 15

        k = jnp.int32(0)
        for t in range(_NW):
            cvec = cv[pl.ds(t * _NW + ((b >> 4) << 4), 16)]
            cnt = _take16(cvec, rotb)[0]
            base = (t * _NW + b) * lcap

            def app(f, k, base=base):
                val = jnp.full((16,), (base + f * g) >> 3, jnp.int32)
                sl = pl.ds((k >> 4) * 16, 16)
                table[sl] = jnp.where(iota16 == (k & 15), val, table[sl])
                return k + 1

            k = lax.fori_loop(0, (cnt + g - 1) // g, app, k)
        kk = k

        def tab_read(i):
            tv = table[pl.ds((i >> 4) * 16, 16)]
            rr = (iota16 + (i & 15)) & 15
            return _take16(tv, rr)[0] * 8

        def issue_lists(j, s):
            off = tab_read(j)
            pltpu.async_copy(ls_hbm.at[pl.ds(off, g)], lvs[s], lsems[s])
            pltpu.async_copy(ld_hbm.at[pl.ds(off, g)], dvs[s], lsems[s])

        def wait_lists(s):
            pltpu.make_async_copy(ls_hbm.at[pl.ds(0, g)], lvs[s],
                                  lsems[s]).wait()
            pltpu.make_async_copy(ld_hbm.at[pl.ds(0, g)], dvs[s],
                                  lsems[s]).wait()

        def issue_gather(sl_, sr):
            pltpu.async_copy(x_hbm.at[lvs[sl_]], rbs[sr], gsems[sr])

        def wait_gather(sl_, sr):
            pltpu.make_async_copy(x_hbm.at[lvs[sl_]], rbs[sr],
                                  gsems[sr]).wait()

        def merge_chunk(sl_, sr):
            def grp(q, _):
                dvec = dvs[sl_][pl.ds(q * 16, 16)] - lo
                for l in range(16):
                    dl = dvec[l]
                    for jj in range(d // 16):
                        cs = pl.ds(jj * 16, 16)
                        acc[dl, cs] = jnp.maximum(acc[dl, cs],
                                                  rbs[sr][q * 16 + l, cs])
                return 0

            lax.fori_loop(0, g // 16, grp, 0)

        for q in range(4):
            @pl.when(q < kk)
            def _(q=q):
                issue_lists(jnp.int32(q), q)

        @pl.when(kk > 0)
        def _():
            wait_lists(0)
            issue_gather(0, 0)

        def quad(i4, _):
            for qq in range(4):
                j = i4 * 4 + qq
                sL, sR = qq, qq & 1
                sL1, sR1 = (qq + 1) % 4, (qq + 1) & 1

                @pl.when(j < kk)
                def _(j=j, sL=sL, sR=sR, sL1=sL1, sR1=sR1):
                    @pl.when(j + 1 < kk)
                    def _():
                        wait_lists(sL1)
                        issue_gather(sL1, sR1)

                    wait_gather(sL, sR)
                    merge_chunk(sL, sR)

                    @pl.when(j + 4 < kk)
                    def _():
                        issue_lists(j + 4, sL)
            return 0

        lax.fori_loop(0, (kk + 3) // 4, quad, 0)

        pltpu.sync_copy(acc.at[pl.ds(0, _NPT)], out_hbm.at[pl.ds(lo, _NPT)])

    return mergek(x, ls, ld, counts)[:n]


def kernel(features, edge_index, W_pool1, b_pool1, W_neigh1, W_self1, b_self1,
           W_pool2, b_pool2, W_neigh2, W_self2, b_self2):
    src = edge_index[0].astype(jnp.int32)
    dst = edge_index[1].astype(jnp.int32)

    (ls, ld, counts), lcap = _scan_edges(src, dst)

    pooled1 = _dense(features, W_pool1.T, b_pool1, relu=True)
    hn1 = _merge(pooled1, ls, ld, counts, lcap)
    h = _combine(features, W_self1.T, b_self1, hn1, W_neigh1.T, relu=True)

    pooled2 = _dense(h, W_pool2.T, b_pool2, relu=True)
    hn2a = _merge(pooled2[:, :128], ls, ld, counts, lcap)
    hn2b = _merge(pooled2[:, 128:], ls, ld, counts, lcap)
    hn2 = jnp.concatenate([hn2a, hn2b], axis=1)
    return _combine(h, W_self2.T, b_self2, hn2, W_neigh2.T, relu=False)


# speedup vs baseline: 1.2887x; 1.2887x over previous
import functools

import jax
import jax.numpy as jnp
from jax import lax
from jax.experimental import pallas as pl
from jax.experimental.pallas import tpu as pltpu
from jax.experimental.pallas import tpu_sc as plsc


_BN = 1000


def _dense(x, w, b, relu):
    n, k = x.shape
    m = w.shape[1]

    def mm(x_ref, w_ref, b_ref, o_ref):
        acc = jnp.dot(x_ref[...], w_ref[...], preferred_element_type=jnp.float32)
        acc = acc + b_ref[...]
        if relu:
            acc = jnp.maximum(acc, 0.0)
        o_ref[...] = acc

    return pl.pallas_call(
        mm,
        grid=(n // _BN,),
        in_specs=[
            pl.BlockSpec((_BN, k), lambda i: (i, 0)),
            pl.BlockSpec((k, m), lambda i: (0, 0)),
            pl.BlockSpec((1, m), lambda i: (0, 0)),
        ],
        out_specs=pl.BlockSpec((_BN, m), lambda i: (i, 0)),
        out_shape=jax.ShapeDtypeStruct((n, m), jnp.float32),
    )(x, w, b.reshape(1, m))


def _combine(x, wx, b, hn, wn, relu):
    n, k = x.shape
    m = wx.shape[1]
    kn = hn.shape[1]

    def mm(x_ref, wx_ref, b_ref, hn_ref, wn_ref, o_ref):
        acc = jnp.dot(x_ref[...], wx_ref[...], preferred_element_type=jnp.float32)
        acc = acc + jnp.dot(hn_ref[...], wn_ref[...],
                            preferred_element_type=jnp.float32)
        acc = acc + b_ref[...]
        if relu:
            acc = jnp.maximum(acc, 0.0)
        o_ref[...] = acc

    return pl.pallas_call(
        mm,
        grid=(n // _BN,),
        in_specs=[
            pl.BlockSpec((_BN, k), lambda i: (i, 0)),
            pl.BlockSpec((k, m), lambda i: (0, 0)),
            pl.BlockSpec((1, m), lambda i: (0, 0)),
            pl.BlockSpec((_BN, kn), lambda i: (i, 0)),
            pl.BlockSpec((kn, m), lambda i: (0, 0)),
        ],
        out_specs=pl.BlockSpec((_BN, m), lambda i: (i, 0)),
        out_shape=jax.ShapeDtypeStruct((n, m), jnp.float32),
    )(x, wx, b.reshape(1, m), hn, wn)


_NC, _NS = 2, 16
_NW = _NC * _NS
_NPT = 320
_BMUL, _BSH = 6554, 21
_CE = 10000
_BLK = 512
_G = 128


def _mesh():
    return plsc.VectorSubcoreMesh(core_axis_name="c", subcore_axis_name="s")


def _take16(v, idx):
    return lax.gather(
        v, idx[:, None],
        lax.GatherDimensionNumbers(offset_dims=(), collapsed_slice_dims=(0,),
                                   start_index_map=(0,)),
        slice_sizes=(1,),
        mode=lax.GatherScatterMode.PROMISE_IN_BOUNDS)


def _scan_edges(src, dst):
    e = src.shape[0]
    ept = e // _NW
    assert ept % _CE == 0 and _CE % 16 == 0
    lcap = ((ept + _BLK - 1) // _BLK) * _BLK
    ltot = _NW * _NW * lcap

    @functools.partial(
        pl.kernel,
        out_type=(jax.ShapeDtypeStruct((ltot,), jnp.int32),
                  jax.ShapeDtypeStruct((ltot,), jnp.int32),
                  jax.ShapeDtypeStruct((_NW * _NW,), jnp.int32)),
        mesh=_mesh(),
        scratch_types=[
            pltpu.VMEM((_NW, _BLK + 16), jnp.int32),
            pltpu.VMEM((_NW, _BLK + 16), jnp.int32),
            pltpu.VMEM((_CE,), jnp.int32),
            pltpu.VMEM((_CE,), jnp.int32),
            pltpu.VMEM((_NW + 16,), jnp.int32),
            pltpu.SMEM((_NW,), jnp.int32),
            pltpu.SMEM((_NW,), jnp.int32),
            pltpu.SemaphoreType.DMA,
        ],
    )
    def scank(src_hbm, dst_hbm, ls_hbm, ld_hbm, cnt_hbm, bs, bd, srcb, dstb,
              cntv, pos, nblk, fsem):
        t = lax.axis_index("s") * _NC + lax.axis_index("c")
        ebase = t * ept
        for b in range(_NW):
            pos[b] = jnp.int32(0)
            nblk[b] = jnp.int32(0)

        def chunk(c, _):
            pltpu.sync_copy(src_hbm.at[pl.ds(ebase + c * _CE, _CE)], srcb)
            pltpu.sync_copy(dst_hbm.at[pl.ds(ebase + c * _CE, _CE)], dstb)

            def grp(i, _):
                s16 = srcb[pl.ds(i * 16, 16)]
                d16 = dstb[pl.ds(i * 16, 16)]
                b16 = (d16 * _BMUL) >> _BSH
                iota16g = lax.iota(jnp.int32, 16)
                for l in range(16):
                    bb = b16[l]
                    c0 = pos[bb]
                    sl = pl.ds((c0 >> 4) * 16, 16)
                    oh = iota16g == (c0 & 15)
                    bs[bb, sl] = jnp.where(oh, jnp.full((16,), s16[l],
                                                        jnp.int32), bs[bb, sl])
                    bd[bb, sl] = jnp.where(oh, jnp.full((16,), d16[l],
                                                        jnp.int32), bd[bb, sl])
                    pos[bb] = c0 + 1

                    def flush(x):
                        base = (t * _NW + bb) * lcap + nblk[bb] * _BLK
                        pltpu.sync_copy(bs.at[bb, pl.ds(0, _BLK)],
                                        ls_hbm.at[pl.ds(base, _BLK)])
                        pltpu.sync_copy(bd.at[bb, pl.ds(0, _BLK)],
                                        ld_hbm.at[pl.ds(base, _BLK)])
                        nblk[bb] = nblk[bb] + 1
                        pos[bb] = jnp.int32(0)
                        return x

                    lax.cond(c0 + 1 >= _BLK, flush, lambda x: x, jnp.int32(0))
                return 0

            lax.fori_loop(0, _CE // 16, grp, 0)
            return 0

        lax.fori_loop(0, ept // _CE, chunk, 0)

        zero16 = jnp.zeros((16,), jnp.int32)

        iota16 = lax.iota(jnp.int32, 16)

        def final_flush(b, _):
            c0 = pos[b]
            dummy = jnp.full((16,), (b + 1) * _NPT, jnp.int32)
            for j in range(_BLK // 16):
                m = (iota16 + j * 16) >= c0
                sl = pl.ds(j * 16, 16)
                bs[b, sl] = jnp.where(m, zero16, bs[b, sl])
                bd[b, sl] = jnp.where(m, dummy, bd[b, sl])
            base = (t * _NW + b) * lcap + nblk[b] * _BLK
            pltpu.sync_copy(bs.at[b, pl.ds(0, _BLK)],
                            ls_hbm.at[pl.ds(base, _BLK)])
            pltpu.sync_copy(bd.at[b, pl.ds(0, _BLK)],
                            ld_hbm.at[pl.ds(base, _BLK)])
            return 0

        lax.fori_loop(0, _NW, final_flush, 0)

        for b in range(_NW - 1, -1, -1):
            cntv[pl.ds(b, 16)] = jnp.full((16,), nblk[b] * _BLK + pos[b],
                                          jnp.int32)
        pltpu.sync_copy(cntv.at[pl.ds(0, _NW)],
                        cnt_hbm.at[pl.ds(t * _NW, _NW)])

    return scank(src, dst), lcap


def _merge(x, ls, ld, counts, lcap):
    n, d = x.shape
    npad = _NW * _NPT

    @functools.partial(
        pl.kernel,
        out_type=jax.ShapeDtypeStruct((npad, d), jnp.float32),
        mesh=_mesh(),
        scratch_types=[
            pltpu.VMEM((_NPT + 1, d), jnp.float32),
            pltpu.VMEM((_G, d), jnp.float32),
            pltpu.VMEM((_G,), jnp.int32),
            pltpu.VMEM((_G,), jnp.int32),
            pltpu.VMEM((_NW * _NW,), jnp.int32),
            pltpu.SemaphoreType.DMA,
        ],
    )
    def mergek(x_hbm, ls_hbm, ld_hbm, cnt_hbm, out_hbm, acc, rows, lv, dv,
               cv, sem):
        b = lax.axis_index("s") * _NC + lax.axis_index("c")
        lo = b * _NPT
        zero16 = jnp.zeros((16,), jnp.float32)

        def zero_row(r, _):
            for j in range(d // 16):
                acc[r, pl.ds(j * 16, 16)] = zero16
            return 0

        lax.fori_loop(0, _NPT + 1, zero_row, 0)
        pltpu.sync_copy(cnt_hbm, cv)
        rot = (lax.iota(jnp.int32, 16) + (b & 15)) & 15

        def sublist(t, _):
            cvec = cv[pl.ds(t * _NW + ((b >> 4) << 4), 16)]
            cnt = _take16(cvec, rot)[0]
            base = (t * _NW + b) * lcap

            def chunk(f, _):
                pltpu.sync_copy(ls_hbm.at[pl.ds(base + f * _G, _G)], lv)
                pltpu.sync_copy(ld_hbm.at[pl.ds(base + f * _G, _G)], dv)
                pltpu.async_copy(x_hbm.at[lv], rows, sem).wait()

                def grp(g, _):
                    dvec = dv[pl.ds(g * 16, 16)] - lo
                    for l in range(16):
                        dl = dvec[l]
                        for j in range(d // 16):
                            sl = pl.ds(j * 16, 16)
                            acc[dl, sl] = jnp.maximum(acc[dl, sl],
                                                      rows[g * 16 + l, sl])
                    return 0

                lax.fori_loop(0, _G // 16, grp, 0)
                return 0

            lax.fori_loop(0, (cnt + _G - 1) // _G, chunk, 0)
            return 0

        lax.fori_loop(0, _NW, sublist, 0)

        pltpu.sync_copy(acc.at[pl.ds(0, _NPT)], out_hbm.at[pl.ds(lo, _NPT)])

    return mergek(x, ls, ld, counts)[:n]


def kernel(features, edge_index, W_pool1, b_pool1, W_neigh1, W_self1, b_self1,
           W_pool2, b_pool2, W_neigh2, W_self2, b_self2):
    src = edge_index[0].astype(jnp.int32)
    dst = edge_index[1].astype(jnp.int32)

    (ls, ld, counts), lcap = _scan_edges(src, dst)

    pooled1 = _dense(features, W_pool1.T, b_pool1, relu=True)
    hn1 = _merge(pooled1, ls, ld, counts, lcap)
    h = _combine(features, W_self1.T, b_self1, hn1, W_neigh1.T, relu=True)

    pooled2 = _dense(h, W_pool2.T, b_pool2, relu=True)
    hn2 = _merge(pooled2, ls, ld, counts, lcap)
    return _combine(h, W_self2.T, b_self2, hn2, W_neigh2.T, relu=False)
